# baseline (speedup 1.0000x reference)
import jax
import jax.numpy as jnp
from jax.experimental import pallas as pl
from jax.experimental.pallas import tpu as pltpu

NUM_SEGMENTS = 16
TOTAL_TOKENS = 32768
DIM = 1024
BLOCK_A = 4096
N_A = 28672
BLOCK_B = 1024
S = NUM_SEGMENTS
NEG = -1e30


def _step(x, a, idr, T, m_ref, d_ref, acc_ref):
    l = jax.lax.dot_general(a, x, (((1,), (1,)), ((), ())),
                            preferred_element_type=jnp.float32)
    seg_st = jax.lax.broadcasted_iota(jnp.int32, (S, T), 0)
    mask = seg_st == idr
    lm = jnp.where(mask, l, NEG)
    bm = jnp.max(lm, axis=1, keepdims=True)
    m_old = m_ref[...]
    m_new = jnp.maximum(m_old, bm)
    c = jnp.exp(m_old - m_new)
    pw = jnp.exp(jnp.where(mask, l - m_new, NEG))
    d_ref[...] = d_ref[...] * c + jnp.sum(pw, axis=1, keepdims=True)
    m_ref[...] = m_new
    acc_ref[...] = (acc_ref[...] * c
                    + jnp.dot(pw, x, preferred_element_type=jnp.float32))


def _body_a(x_ref, att_ref, idr_ref, m_out, d_out, acc_out,
            m_ref, d_ref, acc_ref):
    i = pl.program_id(0)
    nb = pl.num_programs(0)

    @pl.when(i == 0)
    def _init():
        m_ref[...] = jnp.full((S, 1), NEG, jnp.float32)
        d_ref[...] = jnp.zeros((S, 1), jnp.float32)
        acc_ref[...] = jnp.zeros((S, DIM), jnp.float32)

    _step(x_ref[...], att_ref[...], idr_ref[0], BLOCK_A, m_ref, d_ref, acc_ref)

    @pl.when(i == nb - 1)
    def _fin():
        m_out[...] = m_ref[...]
        d_out[...] = d_ref[...]
        acc_out[...] = acc_ref[...]


def _body_b(x_ref, att_ref, idr_ref, m0_ref, d0_ref, acc0_ref, out_ref,
            m_ref, d_ref, acc_ref):
    i = pl.program_id(0)
    nb = pl.num_programs(0)

    @pl.when(i == 0)
    def _init():
        m_ref[...] = m0_ref[...]
        d_ref[...] = d0_ref[...]
        acc_ref[...] = acc0_ref[...]

    _step(x_ref[...], att_ref[...], idr_ref[0], BLOCK_B, m_ref, d_ref, acc_ref)

    @pl.when(i == nb - 1)
    def _fin():
        d = d_ref[...]
        out_ref[...] = jnp.where(d > 0, acc_ref[...] / d, 0.0)


@jax.jit
def kernel(seq, att, segment_ids):
    ids = segment_ids.astype(jnp.int32)
    att_row = att.reshape(1, DIM)

    nba = N_A // BLOCK_A
    idr_a = ids[:N_A].reshape(nba, 1, BLOCK_A)
    m1, d1, acc1 = pl.pallas_call(
        _body_a,
        grid=(nba,),
        in_specs=[
            pl.BlockSpec((BLOCK_A, DIM), lambda i: (i, 0)),
            pl.BlockSpec((1, DIM), lambda i: (0, 0)),
            pl.BlockSpec((1, 1, BLOCK_A), lambda i: (i, 0, 0)),
        ],
        out_specs=[
            pl.BlockSpec((S, 1), lambda i: (0, 0)),
            pl.BlockSpec((S, 1), lambda i: (0, 0)),
            pl.BlockSpec((S, DIM), lambda i: (0, 0)),
        ],
        out_shape=[
            jax.ShapeDtypeStruct((S, 1), jnp.float32),
            jax.ShapeDtypeStruct((S, 1), jnp.float32),
            jax.ShapeDtypeStruct((S, DIM), jnp.float32),
        ],
        scratch_shapes=[
            pltpu.VMEM((S, 1), jnp.float32),
            pltpu.VMEM((S, 1), jnp.float32),
            pltpu.VMEM((S, DIM), jnp.float32),
        ],
        compiler_params=pltpu.CompilerParams(
            dimension_semantics=("arbitrary",)),
    )(seq[:N_A], att_row, idr_a)

    n_b = TOTAL_TOKENS - N_A
    nbb = n_b // BLOCK_B
    idr_b = ids[N_A:].reshape(nbb, 1, BLOCK_B)
    return pl.pallas_call(
        _body_b,
        grid=(nbb,),
        in_specs=[
            pl.BlockSpec((BLOCK_B, DIM), lambda i: (i, 0)),
            pl.BlockSpec((1, DIM), lambda i: (0, 0)),
            pl.BlockSpec((1, 1, BLOCK_B), lambda i: (i, 0, 0)),
            pl.BlockSpec((S, 1), lambda i: (0, 0)),
            pl.BlockSpec((S, 1), lambda i: (0, 0)),
            pl.BlockSpec((S, DIM), lambda i: (0, 0)),
        ],
        out_specs=pl.BlockSpec((S, DIM), lambda i: (0, 0)),
        out_shape=jax.ShapeDtypeStruct((S, DIM), jnp.float32),
        scratch_shapes=[
            pltpu.VMEM((S, 1), jnp.float32),
            pltpu.VMEM((S, 1), jnp.float32),
            pltpu.VMEM((S, DIM), jnp.float32),
        ],
        compiler_params=pltpu.CompilerParams(
            dimension_semantics=("arbitrary",)),
    )(seq[N_A:], att_row, idr_b, m1, d1, acc1)


# speedup vs baseline: 2.4738x; 2.4738x over previous
import jax
import jax.numpy as jnp
from jax.experimental import pallas as pl
from jax.experimental.pallas import tpu as pltpu

NUM_SEGMENTS = 16
TOTAL_TOKENS = 32768
DIM = 1024
BLOCK_A = 4096
N_A = 28672
BLOCK_B = 1024
S = NUM_SEGMENTS
NEG = -1e30


def _step(x, a, idr, T, m_ref, d_ref, acc_ref):
    l = jax.lax.dot_general(a, x, (((1,), (1,)), ((), ())),
                            preferred_element_type=jnp.float32)
    seg_st = jax.lax.broadcasted_iota(jnp.int32, (S, T), 0)
    mask = seg_st == idr
    lm = jnp.where(mask, l, NEG)
    bm = jnp.max(lm, axis=1, keepdims=True)
    m_old = m_ref[...]
    m_new = jnp.maximum(m_old, bm)
    c = jnp.exp(m_old - m_new)
    pw = jnp.exp(jnp.where(mask, l - m_new, NEG))
    d_ref[...] = d_ref[...] * c + jnp.sum(pw, axis=1, keepdims=True)
    m_ref[...] = m_new
    acc_ref[...] = (acc_ref[...] * c
                    + jnp.dot(pw, x, preferred_element_type=jnp.float32))


def _body_a(x_ref, att_ref, idr_ref, m_out, d_out, acc_out,
            m_ref, d_ref, acc_ref):
    i = pl.program_id(0)
    nb = pl.num_programs(0)

    @pl.when(i == 0)
    def _init():
        m_ref[...] = jnp.full((S, 1), NEG, jnp.float32)
        d_ref[...] = jnp.zeros((S, 1), jnp.float32)
        acc_ref[...] = jnp.zeros((S, DIM), jnp.float32)

    _step(x_ref[...], att_ref[...], idr_ref[0], BLOCK_A, m_ref, d_ref, acc_ref)

    @pl.when(i == nb - 1)
    def _fin():
        m_out[...] = m_ref[...]
        d_out[...] = d_ref[...]
        acc_out[...] = acc_ref[...]


def _body_b(x_ref, att_ref, idr_ref, m0_ref, d0_ref, acc0_ref, out_ref,
            m_ref, d_ref, acc_ref):
    i = pl.program_id(0)
    nb = pl.num_programs(0)

    @pl.when(i == 0)
    def _init():
        m_ref[...] = m0_ref[...]
        d_ref[...] = d0_ref[...]
        acc_ref[...] = acc0_ref[...]

    _step(x_ref[...], att_ref[...], idr_ref[0], BLOCK_B, m_ref, d_ref, acc_ref)

    @pl.when(i == nb - 1)
    def _fin():
        d = d_ref[...]
        out_ref[...] = jnp.where(d > 0, acc_ref[...] / d, 0.0)


@jax.jit
def kernel(seq, att, segment_ids):
    ids = segment_ids.astype(jnp.int32)
    att_row = att.reshape(1, DIM)

    nba = N_A // BLOCK_A
    idr_a = ids[:N_A].reshape(nba, 1, BLOCK_A)
    m1, d1, acc1 = pl.pallas_call(
        _body_a,
        grid=(nba,),
        in_specs=[
            pl.BlockSpec((BLOCK_A, DIM), lambda i: (i, 0)),
            pl.BlockSpec((1, DIM), lambda i: (0, 0)),
            pl.BlockSpec((1, 1, BLOCK_A), lambda i: (i, 0, 0)),
        ],
        out_specs=[
            pl.BlockSpec((S, 1), lambda i: (0, 0)),
            pl.BlockSpec((S, 1), lambda i: (0, 0)),
            pl.BlockSpec((S, DIM), lambda i: (0, 0)),
        ],
        out_shape=[
            jax.ShapeDtypeStruct((S, 1), jnp.float32),
            jax.ShapeDtypeStruct((S, 1), jnp.float32),
            jax.ShapeDtypeStruct((S, DIM), jnp.float32),
        ],
        scratch_shapes=[
            pltpu.VMEM((S, 1), jnp.float32),
            pltpu.VMEM((S, 1), jnp.float32),
            pltpu.VMEM((S, DIM), jnp.float32),
        ],
        compiler_params=pltpu.CompilerParams(
            dimension_semantics=("arbitrary",)),
    )(seq, att_row, idr_a)

    n_b = TOTAL_TOKENS - N_A
    nbb = n_b // BLOCK_B
    idr_b = ids[N_A:].reshape(nbb, 1, BLOCK_B)
    return pl.pallas_call(
        _body_b,
        grid=(nbb,),
        in_specs=[
            pl.BlockSpec((BLOCK_B, DIM), lambda i: (i + N_A // BLOCK_B, 0)),
            pl.BlockSpec((1, DIM), lambda i: (0, 0)),
            pl.BlockSpec((1, 1, BLOCK_B), lambda i: (i, 0, 0)),
            pl.BlockSpec((S, 1), lambda i: (0, 0)),
            pl.BlockSpec((S, 1), lambda i: (0, 0)),
            pl.BlockSpec((S, DIM), lambda i: (0, 0)),
        ],
        out_specs=pl.BlockSpec((S, DIM), lambda i: (0, 0)),
        out_shape=jax.ShapeDtypeStruct((S, DIM), jnp.float32),
        scratch_shapes=[
            pltpu.VMEM((S, 1), jnp.float32),
            pltpu.VMEM((S, 1), jnp.float32),
            pltpu.VMEM((S, DIM), jnp.float32),
        ],
        compiler_params=pltpu.CompilerParams(
            dimension_semantics=("arbitrary",)),
    )(seq, att_row, idr_b, m1, d1, acc1)


# speedup vs baseline: 2.8884x; 1.1676x over previous
import functools

import jax
import jax.numpy as jnp
from jax.experimental import pallas as pl
from jax.experimental.pallas import tpu as pltpu

NUM_SEGMENTS = 16
TOTAL_TOKENS = 32768
DIM = 1024
BLOCK_T = 4096
NEG = -1e30


def _body(x_ref, att_ref, idr_ref, out_ref, m_ref, d_ref, acc_ref):
    i = pl.program_id(0)
    nb = pl.num_programs(0)
    S = NUM_SEGMENTS
    T = BLOCK_T

    @pl.when(i == 0)
    def _init():
        m_ref[...] = jnp.full((S, 1), NEG, jnp.float32)
        d_ref[...] = jnp.zeros((S, 1), jnp.float32)
        acc_ref[...] = jnp.zeros((S, DIM), jnp.float32)

    x = x_ref[...]
    a = att_ref[...]
    idr = idr_ref[0]

    l = jax.lax.dot_general(a, x, (((1,), (1,)), ((), ())),
                            preferred_element_type=jnp.float32)

    seg_st = jax.lax.broadcasted_iota(jnp.int32, (S, T), 0)
    mask = seg_st == idr
    lm = jnp.where(mask, l, NEG)
    bm = jnp.max(lm, axis=1, keepdims=True)
    m_old = m_ref[...]
    m_new = jnp.maximum(m_old, bm)
    c = jnp.exp(m_old - m_new)
    pw = jnp.exp(jnp.where(mask, l - m_new, NEG))
    d_ref[...] = d_ref[...] * c + jnp.sum(pw, axis=1, keepdims=True)
    m_ref[...] = m_new
    acc_ref[...] = (acc_ref[...] * c
                    + jnp.dot(pw, x, preferred_element_type=jnp.float32))

    @pl.when(i == nb - 1)
    def _fin():
        d = d_ref[...]
        out_ref[...] = jnp.where(d > 0, acc_ref[...] / d, 0.0)


@jax.jit
def kernel(seq, att, segment_ids):
    ids = segment_ids.astype(jnp.int32)
    nb = TOTAL_TOKENS // BLOCK_T
    idr = ids.reshape(nb, 1, BLOCK_T)
    att_row = att.reshape(1, DIM)
    return pl.pallas_call(
        _body,
        grid=(nb,),
        in_specs=[
            pl.BlockSpec((BLOCK_T, DIM), lambda i: (i, 0)),
            pl.BlockSpec((1, DIM), lambda i: (0, 0)),
            pl.BlockSpec((1, 1, BLOCK_T), lambda i: (i, 0, 0)),
        ],
        out_specs=pl.BlockSpec((NUM_SEGMENTS, DIM), lambda i: (0, 0)),
        out_shape=jax.ShapeDtypeStruct((NUM_SEGMENTS, DIM), jnp.float32),
        scratch_shapes=[
            pltpu.VMEM((NUM_SEGMENTS, 1), jnp.float32),
            pltpu.VMEM((NUM_SEGMENTS, 1), jnp.float32),
            pltpu.VMEM((NUM_SEGMENTS, DIM), jnp.float32),
        ],
        compiler_params=pltpu.CompilerParams(
            dimension_semantics=("arbitrary",)),
    )(seq, att_row, idr)
